# baseline (speedup 1.0000x reference)
import functools

import jax
import jax.numpy as jnp
from jax import lax
from jax.experimental import pallas as pl
from jax.experimental.pallas import tpu as pltpu
from jax.experimental.pallas import tpu_sc as plsc

_RANGE_MIN = 24
_RANGE_MAX = 31
_IN_DIM = 128
_EMB_DIM = 18
_B = 16384
_NW = 32
_ROWS = _B // _NW
_CHUNK = _ROWS * _IN_DIM
_NCARD = _RANGE_MAX - _RANGE_MIN
_NFIX = _ROWS * _NCARD
_NGRP = _NFIX // 16


def _sc_body(x_hbm, ft_hbm, out_hbm, pv, ft_v):
    nc = plsc.get_sparse_core_info().num_cores
    wid = lax.axis_index("s") * nc + lax.axis_index("c")
    base = wid * _CHUNK
    pltpu.sync_copy(ft_hbm, ft_v)
    for e in range(_EMB_DIM):
        pltpu.sync_copy(x_hbm.at[pl.ds(base, _CHUNK)], pv)

        def fix(g, carry):
            fi = ft_v[pl.ds(g * 16, 16)]
            if e == _EMB_DIM - 1:
                bitf = jnp.ones((16,), jnp.float32)
            else:
                vals = plsc.load_gather(pv, [fi])
                vi = vals.astype(jnp.int32)
                if e < 13:
                    hit = (vi >> 2) == e
                else:
                    hit = (vi & 3) == (e - 13)
                bitf = jnp.where(hit, 1.0, 0.0).astype(jnp.float32)
            plsc.store_scatter(pv, [fi], bitf)
            return carry

        lax.fori_loop(0, _NGRP, fix, 0)
        pltpu.sync_copy(pv, out_hbm.at[e, pl.ds(base, _CHUNK)])


@jax.jit
def _run(x2):
    f = jnp.arange(_NFIX, dtype=jnp.int32)
    flat_tbl = (f // _NCARD) * _IN_DIM + _RANGE_MIN + f % _NCARD
    mesh = plsc.VectorSubcoreMesh(core_axis_name="c", subcore_axis_name="s")
    k = functools.partial(
        pl.kernel,
        mesh=mesh,
        out_type=jax.ShapeDtypeStruct((_EMB_DIM, _B * _IN_DIM), jnp.float32),
        scratch_types=[
            pltpu.VMEM((_CHUNK,), jnp.float32),
            pltpu.VMEM((_NFIX,), jnp.int32),
        ],
        compiler_params=pltpu.CompilerParams(needs_layout_passes=False),
    )(_sc_body)
    out = k(x2.reshape(_B * _IN_DIM), flat_tbl)
    return out.reshape(_EMB_DIM, _B, _IN_DIM).transpose(1, 2, 0)


def kernel(x):
    if x.ndim == 3:
        x = x[:, 0, :]
    return _run(x)


# speedup vs baseline: 8.3368x; 8.3368x over previous
import jax
import jax.numpy as jnp
from jax.experimental import pallas as pl

_RANGE_MIN = 24
_RANGE_MAX = 31
_IN_DIM = 128
_EMB_DIM = 18


def _body(x_ref, o_ref):
    v = x_ref[...]
    ci = jax.lax.broadcasted_iota(jnp.int32, v.shape, 1)
    is_card = (ci >= _RANGE_MIN) & (ci < _RANGE_MAX)
    vi = jnp.floor(v)
    r = jnp.floor(vi * 0.25)
    s = vi - 4.0 * r
    one = jnp.ones_like(v)
    zero = jnp.zeros_like(v)
    for e in range(_EMB_DIM):
        if e < 13:
            bit = jnp.where(r == float(e), one, zero)
        elif e < 17:
            bit = jnp.where(s == float(e - 13), one, zero)
        else:
            bit = one
        o_ref[e] = jnp.where(is_card, bit, v)


@jax.jit
def _run(x2):
    b = x2.shape[0]
    bb = 2048
    out = pl.pallas_call(
        _body,
        grid=(b // bb,),
        in_specs=[pl.BlockSpec((bb, _IN_DIM), lambda i: (i, 0))],
        out_specs=pl.BlockSpec((_EMB_DIM, bb, _IN_DIM), lambda i: (0, i, 0)),
        out_shape=jax.ShapeDtypeStruct((_EMB_DIM, b, _IN_DIM), jnp.float32),
    )(x2)
    return out.transpose(1, 2, 0)


def kernel(x):
    if x.ndim == 3:
        x = x[:, 0, :]
    return _run(x)
